# Initial kernel scaffold; baseline (speedup 1.0000x reference)
import jax
import jax.numpy as jnp
from jax.experimental import pallas as pl


def kernel(length_q, length_k, pe):
    raise NotImplementedError("write your pallas kernel here")


# speedup vs baseline: 2.9492x; 2.9492x over previous
import jax
import jax.numpy as jnp
from jax import lax
from jax.experimental import pallas as pl
from jax.experimental.pallas import tpu as pltpu
from jax.experimental.pallas import tpu_sc as plsc

L_Q = 1024
L_K = 1024
D_MODEL = 64
MAX_K = 16
N_TILES = 32
Q_PER_TILE = L_Q // N_TILES
N_CHUNKS = 9
W_ROWS = N_CHUNKS * 128


def _sc_body(pe_hbm, out_hbm, idx_v, win_v, sem):
    c = lax.axis_index("c")
    s = lax.axis_index("s")
    wid = s * 2 + c
    q0 = wid * Q_PER_TILE
    base = (L_K - 1) - (q0 + Q_PER_TILE - 1)

    for t in range(N_CHUNKS):
        for o in range(8):
            g = lax.iota(jnp.int32, 16) + (t * 128 + o * 16)
            rel = g + base - (L_K - 1)
            idx_v[t, pl.ds(o * 16, 16)] = jnp.clip(rel, -MAX_K, MAX_K) + MAX_K

    gathers = [
        pltpu.async_copy(pe_hbm.at[idx_v.at[t]],
                         win_v.at[pl.ds(t * 128, 128)], sem)
        for t in range(N_CHUNKS)
    ]
    for g in gathers:
        g.wait()

    writes = [
        pltpu.async_copy(win_v.at[pl.ds(Q_PER_TILE - 1 - j, L_K)],
                         out_hbm.at[q0 + j], sem)
        for j in range(Q_PER_TILE)
    ]
    for w in writes:
        w.wait()


def kernel(length_q, length_k, pe):
    del length_q, length_k
    mesh = plsc.VectorSubcoreMesh(core_axis_name="c", subcore_axis_name="s")
    f = pl.kernel(
        _sc_body,
        out_type=jax.ShapeDtypeStruct((L_Q, L_K, D_MODEL), jnp.float32),
        mesh=mesh,
        scratch_types=[
            pltpu.VMEM((N_CHUNKS, 128), jnp.int32),
            pltpu.VMEM((W_ROWS, D_MODEL), jnp.float32),
            pltpu.SemaphoreType.DMA,
        ],
        compiler_params=pltpu.CompilerParams(use_tc_tiling_on_sc=False),
    )
    return f(pe)


# speedup vs baseline: 5.4523x; 1.8487x over previous
import jax
import jax.numpy as jnp
from jax import lax
from jax.experimental import pallas as pl
from jax.experimental.pallas import tpu as pltpu
from jax.experimental.pallas import tpu_sc as plsc

L_Q = 1024
L_K = 1024
D_MODEL = 64
MAX_K = 16
N_TILES = 32
Q_PER_TILE = L_Q // N_TILES
W_ROWS = L_K + Q_PER_TILE
ROW_ELEMS = L_K * D_MODEL


def _sc_body(pe_hbm, out_hbm, pe_v, win_v, sem):
    c = lax.axis_index("c")
    s = lax.axis_index("s")
    wid = s * 2 + c
    q0 = wid * Q_PER_TILE
    base = (L_K - 1) - (q0 + Q_PER_TILE - 1)

    pltpu.sync_copy(pe_hbm, pe_v)

    def fill(r, carry):
        rel = r + base - (L_K - 1)
        sidx = jnp.clip(rel, -MAX_K, MAX_K) + MAX_K
        src = sidx * D_MODEL
        dst = r * D_MODEL
        for u in range(D_MODEL // 16):
            win_v[pl.ds(dst + u * 16, 16)] = pe_v[pl.ds(src + u * 16, 16)]
        return carry

    lax.fori_loop(0, W_ROWS, fill, 0)

    writes = [
        pltpu.async_copy(
            win_v.at[pl.ds((Q_PER_TILE - 1 - j) * D_MODEL, ROW_ELEMS)],
            out_hbm.at[pl.ds((q0 + j) * ROW_ELEMS, ROW_ELEMS)],
            sem,
        )
        for j in range(Q_PER_TILE)
    ]
    for w in writes:
        w.wait()


def kernel(length_q, length_k, pe):
    del length_q, length_k
    mesh = plsc.VectorSubcoreMesh(core_axis_name="c", subcore_axis_name="s")
    f = pl.kernel(
        _sc_body,
        out_type=jax.ShapeDtypeStruct((L_Q * ROW_ELEMS,), jnp.float32),
        mesh=mesh,
        scratch_types=[
            pltpu.VMEM(((2 * MAX_K + 1) * D_MODEL,), jnp.float32),
            pltpu.VMEM((W_ROWS * D_MODEL,), jnp.float32),
            pltpu.SemaphoreType.DMA,
        ],
        compiler_params=pltpu.CompilerParams(use_tc_tiling_on_sc=False),
    )
    out_flat = f(pe.reshape(-1))
    return out_flat.reshape(L_Q, L_K, D_MODEL)


# speedup vs baseline: 9.7678x; 1.7915x over previous
import jax
import jax.numpy as jnp
from jax import lax
from jax.experimental import pallas as pl
from jax.experimental.pallas import tpu as pltpu
from jax.experimental.pallas import tpu_sc as plsc

L_Q = 1024
L_K = 1024
D_MODEL = 64
MAX_K = 16
N_PE = 2 * MAX_K + 1
N_TILES = 32
Q_PER_TILE = L_Q // N_TILES
W_COLS = 1280


def _sc_body(pet_hbm, out_hbm, pet_v, win_v, sem):
    c = lax.axis_index("c")
    s = lax.axis_index("s")
    wid = s * 2 + c
    r = wid % 8
    a = wid // 8
    q_max = r + 256 * a + 8 * (Q_PER_TILE - 1)

    pltpu.sync_copy(pet_hbm, pet_v)

    def fill_d(d, carry):
        row = d * N_PE

        def fill_c(cc, carry2):
            rel = lax.iota(jnp.int32, 16) + cc * 16 - q_max
            sidx = jnp.clip(rel, -MAX_K, MAX_K) + MAX_K
            win_v[d, pl.ds(cc * 16, 16)] = plsc.load_gather(pet_v, [row + sidx])
            return carry2

        lax.fori_loop(0, W_COLS // 16, fill_c, 0)
        return carry

    lax.fori_loop(0, D_MODEL, fill_d, 0)

    writes = [
        pltpu.async_copy(
            win_v.at[:, pl.ds(8 * (Q_PER_TILE - 1 - i), L_K)],
            out_hbm.at[r + 256 * a + 8 * i],
            sem,
        )
        for i in range(Q_PER_TILE)
    ]
    for w in writes:
        w.wait()


def kernel(length_q, length_k, pe):
    del length_q, length_k
    mesh = plsc.VectorSubcoreMesh(core_axis_name="c", subcore_axis_name="s")
    f = pl.kernel(
        _sc_body,
        out_type=jax.ShapeDtypeStruct((L_Q, D_MODEL, L_K), jnp.float32),
        mesh=mesh,
        scratch_types=[
            pltpu.VMEM((D_MODEL * N_PE,), jnp.float32),
            pltpu.VMEM((D_MODEL, W_COLS), jnp.float32),
            pltpu.SemaphoreType.DMA,
        ],
        compiler_params=pltpu.CompilerParams(
            use_tc_tiling_on_sc=False, needs_layout_passes=False
        ),
    )
    out_qdk = f(pe.T.reshape(-1))
    return jnp.transpose(out_qdk, (0, 2, 1))


# speedup vs baseline: 24.1489x; 2.4723x over previous
import jax
import jax.numpy as jnp
from jax import lax
from jax.experimental import pallas as pl
from jax.experimental.pallas import tpu as pltpu
from jax.experimental.pallas import tpu_sc as plsc

L_Q = 1024
L_K = 1024
D_MODEL = 64
MAX_K = 16
N_PE = 2 * MAX_K + 1
N_TILES = 32
D_PER_TILE = D_MODEL // N_TILES
ET_COLS = 2176
BQ = 8


def _sc_gather_body(pet_hbm, et_hbm, pet_v, row_v, sem):
    c = lax.axis_index("c")
    s = lax.axis_index("s")
    wid = s * 2 + c

    pltpu.sync_copy(pet_hbm, pet_v)

    for dd in range(D_PER_TILE):
        d = wid * D_PER_TILE + dd
        row = d * N_PE

        def fill(cc, carry, row=row, dd=dd):
            g = lax.iota(jnp.int32, 16) + cc * 16
            sidx = jnp.clip(g - (L_K - 1), -MAX_K, MAX_K) + MAX_K
            row_v[dd, pl.ds(cc * 16, 16)] = plsc.load_gather(pet_v, [row + sidx])
            return carry

        lax.fori_loop(0, ET_COLS // 16, fill, 0)

    writes = [
        pltpu.async_copy(row_v.at[dd], et_hbm.at[wid * D_PER_TILE + dd], sem)
        for dd in range(D_PER_TILE)
    ]
    for w in writes:
        w.wait()


def _tc_window_body(e_ref, out_ref):
    i = pl.program_id(0)
    for j in range(BQ):
        q = i * BQ + j
        off = (L_K - 1) - q
        base = pl.multiple_of((off // 128) * 128, 128)
        m = off % 128
        w = e_ref[:, pl.ds(base, L_K + 128)]
        rolled = pltpu.roll(w, -m, 1)
        out_ref[j] = rolled[:, :L_K]


def kernel(length_q, length_k, pe):
    del length_q, length_k

    mesh = plsc.VectorSubcoreMesh(core_axis_name="c", subcore_axis_name="s")
    gather = pl.kernel(
        _sc_gather_body,
        out_type=jax.ShapeDtypeStruct((D_MODEL, ET_COLS), jnp.float32),
        mesh=mesh,
        scratch_types=[
            pltpu.VMEM((D_MODEL * N_PE,), jnp.float32),
            pltpu.VMEM((D_PER_TILE, ET_COLS), jnp.float32),
            pltpu.SemaphoreType.DMA,
        ],
        compiler_params=pltpu.CompilerParams(
            use_tc_tiling_on_sc=False, needs_layout_passes=False
        ),
    )
    et = gather(pe.T.reshape(-1))

    out_qdk = pl.pallas_call(
        _tc_window_body,
        grid=(L_Q // BQ,),
        in_specs=[pl.BlockSpec((D_MODEL, ET_COLS), lambda i: (0, 0))],
        out_specs=pl.BlockSpec((BQ, D_MODEL, L_K), lambda i: (i, 0, 0)),
        out_shape=jax.ShapeDtypeStruct((L_Q, D_MODEL, L_K), jnp.float32),
    )(et)
    return jnp.transpose(out_qdk, (0, 2, 1))
